# baseline (speedup 1.0000x reference)
import jax
import jax.numpy as jnp
from jax import lax
from jax.experimental import pallas as pl
from jax.experimental.pallas import tpu as pltpu
from jax.experimental.pallas import tpu_sc as plsc

N = 10000
E = 320000
D = 128
NC = 2
NS = 16
NW = NC * NS
CH = 128
SB = 16
SBN = 5
E2 = NW * SBN * SB * CH
NP = 10240
RPT = NP // NS

_MESH = plsc.VectorSubcoreMesh(core_axis_name="c", subcore_axis_name="s",
                               num_cores=NC, num_subcores=NS)


def _sc_agg_body(x_hbm, row_hbm, col_hbm, z_hbm,
                 agg_out, row_v, col_v, rows_a, acc_sh):
    c = lax.axis_index("c")
    s = lax.axis_index("s")
    wid = s * NC + c

    pltpu.sync_copy(z_hbm, rows_a)

    @pl.loop(0, RPT // CH)
    def _(k):
        pltpu.sync_copy(rows_a, acc_sh.at[pl.ds(s * RPT + k * CH, CH)])

    plsc.subcore_barrier()

    @pl.loop(0, SBN)
    def _(bk):
        pltpu.sync_copy(row_hbm.at[wid * SBN + bk], row_v)
        pltpu.sync_copy(col_hbm.at[wid * SBN + bk], col_v)

        @pl.loop(0, SB)
        def _(i):
            pltpu.sync_copy(x_hbm.at[row_v.at[i]], rows_a)
            pltpu.sync_copy(rows_a, acc_sh.at[col_v.at[i]], add=True)

    plsc.subcore_barrier()
    off = c * NP + s * RPT
    pltpu.sync_copy(acc_sh.at[pl.ds(s * RPT, RPT)], agg_out.at[pl.ds(off, RPT)])


_sc_agg = pl.kernel(
    _sc_agg_body,
    out_type=[jax.ShapeDtypeStruct((NC * NP, D), jnp.float32)],
    mesh=_MESH,
    scratch_types=[
        pltpu.VMEM((SB, CH), jnp.int32),
        pltpu.VMEM((SB, CH), jnp.int32),
        pltpu.VMEM((CH, D), jnp.float32),
        pltpu.VMEM_SHARED((NP, D), jnp.float32),
    ],
)


def _sc_deg_body(col_hbm, z_hbm, ones_hbm, deg_out, col_v, ones_v, rows_a,
                 acc_sh):
    c = lax.axis_index("c")
    s = lax.axis_index("s")
    wid = s * NC + c

    pltpu.sync_copy(z_hbm, rows_a)
    pltpu.sync_copy(ones_hbm, ones_v)

    @pl.loop(0, RPT // CH)
    def _(k):
        pltpu.sync_copy(rows_a, acc_sh.at[pl.ds(s * RPT + k * CH, CH)])

    plsc.subcore_barrier()

    @pl.loop(0, SBN)
    def _(bk):
        pltpu.sync_copy(col_hbm.at[wid * SBN + bk], col_v)

        @pl.loop(0, SB)
        def _(i):
            pltpu.sync_copy(ones_v, acc_sh.at[col_v.at[i]], add=True)

    plsc.subcore_barrier()
    off = c * NP + s * RPT
    pltpu.sync_copy(acc_sh.at[pl.ds(s * RPT, RPT)], deg_out.at[pl.ds(off, RPT)])


_sc_deg = pl.kernel(
    _sc_deg_body,
    out_type=[jax.ShapeDtypeStruct((NC * NP, D), jnp.float32)],
    mesh=_MESH,
    scratch_types=[
        pltpu.VMEM((SB, CH), jnp.int32),
        pltpu.VMEM((CH, D), jnp.float32),
        pltpu.VMEM((CH, D), jnp.float32),
        pltpu.VMEM_SHARED((NP, D), jnp.float32),
    ],
)

_BN = 2000


def _tc_layer_body(p0, p1, d0, d1, x, w, b, o):
    deg = jnp.maximum(d0[:, :1] + d1[:, :1], 1.0)
    t = (p0[...] + p1[...]) / deg + x[...]
    acc = jnp.dot(t, w[...], preferred_element_type=jnp.float32,
                  precision=lax.Precision.HIGHEST)
    o[...] = jnp.maximum(acc + b[...], 0.0)


def _tc_layer(agg, deg, x, Wt, b2d, dout):
    grid = (N // _BN,)
    specs = [
        pl.BlockSpec((_BN, D), lambda i: (i, 0)),
        pl.BlockSpec((_BN, D), lambda i: (i, 0)),
        pl.BlockSpec((_BN, D), lambda i: (i, 0)),
        pl.BlockSpec((_BN, D), lambda i: (i, 0)),
        pl.BlockSpec((_BN, D), lambda i: (i, 0)),
        pl.BlockSpec((D, dout), lambda i: (0, 0)),
        pl.BlockSpec((1, dout), lambda i: (0, 0)),
    ]
    return pl.pallas_call(
        _tc_layer_body, grid=grid, in_specs=specs,
        out_specs=pl.BlockSpec((_BN, dout), lambda i: (i, 0)),
        out_shape=jax.ShapeDtypeStruct((N, dout), jnp.float32),
    )(agg[:N], agg[NP:NP + N], deg[:N], deg[NP:NP + N], x, Wt, b2d)


def _tc_final_body(p0, p1, d0, d1, x, w3, b3, wp, bp, wc, bc,
                   emb, ne, ec):
    deg = jnp.maximum(d0[:, :1] + d1[:, :1], 1.0)
    t = (p0[...] + p1[...]) / deg + x[...]
    h = jnp.maximum(
        jnp.dot(t, w3[...], preferred_element_type=jnp.float32,
                precision=lax.Precision.HIGHEST) + b3[...], 0.0)
    emb[...] = h
    ne[...] = jnp.dot(h, wp[...], preferred_element_type=jnp.float32,
                      precision=lax.Precision.HIGHEST) + bp[...]
    ec[...] = jnp.dot(h, wc[...], preferred_element_type=jnp.float32,
                      precision=lax.Precision.HIGHEST) + bc[...]


def _tc_final(agg, deg, x, W3t, b3_2d, Wpt, bp2d, Wct, bc2d):
    grid = (N // _BN,)
    dh = W3t.shape[1]
    dp = Wpt.shape[1]
    dc = Wct.shape[1]
    specs = [
        pl.BlockSpec((_BN, D), lambda i: (i, 0)),
        pl.BlockSpec((_BN, D), lambda i: (i, 0)),
        pl.BlockSpec((_BN, D), lambda i: (i, 0)),
        pl.BlockSpec((_BN, D), lambda i: (i, 0)),
        pl.BlockSpec((_BN, D), lambda i: (i, 0)),
        pl.BlockSpec((D, dh), lambda i: (0, 0)),
        pl.BlockSpec((1, dh), lambda i: (0, 0)),
        pl.BlockSpec((dh, dp), lambda i: (0, 0)),
        pl.BlockSpec((1, dp), lambda i: (0, 0)),
        pl.BlockSpec((dh, dc), lambda i: (0, 0)),
        pl.BlockSpec((1, dc), lambda i: (0, 0)),
    ]
    return pl.pallas_call(
        _tc_final_body, grid=grid, in_specs=specs,
        out_specs=[
            pl.BlockSpec((_BN, dh), lambda i: (i, 0)),
            pl.BlockSpec((_BN, dp), lambda i: (i, 0)),
            pl.BlockSpec((_BN, dc), lambda i: (i, 0)),
        ],
        out_shape=[
            jax.ShapeDtypeStruct((N, dh), jnp.float32),
            jax.ShapeDtypeStruct((N, dp), jnp.float32),
            jax.ShapeDtypeStruct((N, dc), jnp.float32),
        ],
    )(agg[:N], agg[NP:NP + N], deg[:N], deg[NP:NP + N], x, W3t, b3_2d, Wpt,
      bp2d, Wct, bc2d)


def kernel(node_features, edge_index, W1, b1, W2, b2, W3, b3, Wp, bp, Wc, bc):
    npad = E2 - E
    pad_src = jnp.arange(npad, dtype=edge_index.dtype) % N
    row_p = jnp.concatenate([edge_index[0], pad_src])
    pad_dst = (N + jnp.arange(npad, dtype=edge_index.dtype) % (NP - N))
    col_p = jnp.concatenate([edge_index[1], pad_dst])
    row = row_p.reshape(NW * SBN, SB, CH)
    col = col_p.reshape(NW * SBN, SB, CH)
    z = jnp.zeros((CH, D), jnp.float32)
    ones = jnp.ones((CH, D), jnp.float32)

    (deg,) = _sc_deg(col, z, ones)
    (agg,) = _sc_agg(node_features, row, col, z)
    h1 = _tc_layer(agg, deg, node_features, W1.T, b1.reshape(1, -1), D)
    (agg,) = _sc_agg(h1, row, col, z)
    h2 = _tc_layer(agg, deg, h1, W2.T, b2.reshape(1, -1), D)
    (agg,) = _sc_agg(h2, row, col, z)

    n_cls = Wc.shape[0]
    Wct = jnp.zeros((W3.shape[0], 128), jnp.float32).at[:, :n_cls].set(Wc.T)
    bc_pad = jnp.zeros((1, 128), jnp.float32).at[:, :n_cls].set(bc)
    emb, ne, ec_pad = _tc_final(agg, deg, h2, W3.T, b3.reshape(1, -1),
                                Wp.T, bp.reshape(1, -1), Wct, bc_pad)
    return emb, ne, ec_pad[:, :n_cls]


# speedup vs baseline: 1.3368x; 1.3368x over previous
import jax
import jax.numpy as jnp
from jax import lax
from jax.experimental import pallas as pl
from jax.experimental.pallas import tpu as pltpu
from jax.experimental.pallas import tpu_sc as plsc

N = 10000
E = 320000
D = 128
NC = 2
NS = 16
NW = NC * NS
CH = 128
SB = 16
SBN = 5
E2 = NW * SBN * SB * CH
NP = 10240
RPT = NP // NS

_MESH = plsc.VectorSubcoreMesh(core_axis_name="c", subcore_axis_name="s",
                               num_cores=NC, num_subcores=NS)


def _sc_agg_body(x_hbm, row_hbm, col_hbm, z_hbm,
                 agg_out, row_v, col_v, buf0, buf1, acc_sh, sem0, sem1):
    c = lax.axis_index("c")
    s = lax.axis_index("s")
    wid = s * NC + c

    pltpu.sync_copy(z_hbm, buf0)

    @pl.loop(0, RPT // CH)
    def _(k):
        pltpu.sync_copy(buf0, acc_sh.at[pl.ds(s * RPT + k * CH, CH)])

    plsc.subcore_barrier()

    @pl.loop(0, SBN)
    def _(bk):
        pltpu.sync_copy(row_hbm.at[wid * SBN + bk], row_v)
        pltpu.sync_copy(col_hbm.at[wid * SBN + bk], col_v)

        pltpu.async_copy(x_hbm.at[row_v.at[0]], buf0, sem0)

        @pl.loop(0, SB // 2 - 1)
        def _(j):
            i = 2 * j
            pltpu.async_copy(x_hbm.at[row_v.at[i + 1]], buf1, sem1)
            pltpu.make_async_copy(x_hbm.at[row_v.at[i]], buf0, sem0).wait()
            pltpu.sync_copy(buf0, acc_sh.at[col_v.at[i]], add=True)
            pltpu.async_copy(x_hbm.at[row_v.at[i + 2]], buf0, sem0)
            pltpu.make_async_copy(x_hbm.at[row_v.at[i + 1]], buf1, sem1).wait()
            pltpu.sync_copy(buf1, acc_sh.at[col_v.at[i + 1]], add=True)

        pltpu.async_copy(x_hbm.at[row_v.at[SB - 1]], buf1, sem1)
        pltpu.make_async_copy(x_hbm.at[row_v.at[SB - 2]], buf0, sem0).wait()
        pltpu.sync_copy(buf0, acc_sh.at[col_v.at[SB - 2]], add=True)
        pltpu.make_async_copy(x_hbm.at[row_v.at[SB - 1]], buf1, sem1).wait()
        pltpu.sync_copy(buf1, acc_sh.at[col_v.at[SB - 1]], add=True)

    plsc.subcore_barrier()
    off = c * NP + s * RPT
    pltpu.sync_copy(acc_sh.at[pl.ds(s * RPT, RPT)], agg_out.at[pl.ds(off, RPT)])


_sc_agg = pl.kernel(
    _sc_agg_body,
    out_type=[jax.ShapeDtypeStruct((NC * NP, D), jnp.float32)],
    mesh=_MESH,
    scratch_types=[
        pltpu.VMEM((SB, CH), jnp.int32),
        pltpu.VMEM((SB, CH), jnp.int32),
        pltpu.VMEM((CH, D), jnp.float32),
        pltpu.VMEM((CH, D), jnp.float32),
        pltpu.VMEM_SHARED((NP, D), jnp.float32),
        pltpu.SemaphoreType.DMA,
        pltpu.SemaphoreType.DMA,
    ],
)


def _sc_deg_body(col_hbm, z_hbm, ones_hbm, deg_out, col_v, ones_v, rows_a,
                 acc_sh):
    c = lax.axis_index("c")
    s = lax.axis_index("s")
    wid = s * NC + c

    pltpu.sync_copy(z_hbm, rows_a)
    pltpu.sync_copy(ones_hbm, ones_v)

    @pl.loop(0, RPT // CH)
    def _(k):
        pltpu.sync_copy(rows_a, acc_sh.at[pl.ds(s * RPT + k * CH, CH)])

    plsc.subcore_barrier()

    @pl.loop(0, SBN)
    def _(bk):
        pltpu.sync_copy(col_hbm.at[wid * SBN + bk], col_v)

        @pl.loop(0, SB)
        def _(i):
            pltpu.sync_copy(ones_v, acc_sh.at[col_v.at[i]], add=True)

    plsc.subcore_barrier()
    off = c * NP + s * RPT
    pltpu.sync_copy(acc_sh.at[pl.ds(s * RPT, RPT)], deg_out.at[pl.ds(off, RPT)])


_sc_deg = pl.kernel(
    _sc_deg_body,
    out_type=[jax.ShapeDtypeStruct((NC * NP, D), jnp.float32)],
    mesh=_MESH,
    scratch_types=[
        pltpu.VMEM((SB, CH), jnp.int32),
        pltpu.VMEM((CH, D), jnp.float32),
        pltpu.VMEM((CH, D), jnp.float32),
        pltpu.VMEM_SHARED((NP, D), jnp.float32),
    ],
)

_BN = 2000


def _tc_layer_body(p0, p1, d0, d1, x, w, b, o):
    deg = jnp.maximum(d0[:, :1] + d1[:, :1], 1.0)
    t = (p0[...] + p1[...]) / deg + x[...]
    acc = jnp.dot(t, w[...], preferred_element_type=jnp.float32,
                  precision=lax.Precision.HIGHEST)
    o[...] = jnp.maximum(acc + b[...], 0.0)


def _tc_layer(agg, deg, x, Wt, b2d, dout):
    grid = (N // _BN,)
    specs = [
        pl.BlockSpec((_BN, D), lambda i: (i, 0)),
        pl.BlockSpec((_BN, D), lambda i: (i, 0)),
        pl.BlockSpec((_BN, D), lambda i: (i, 0)),
        pl.BlockSpec((_BN, D), lambda i: (i, 0)),
        pl.BlockSpec((_BN, D), lambda i: (i, 0)),
        pl.BlockSpec((D, dout), lambda i: (0, 0)),
        pl.BlockSpec((1, dout), lambda i: (0, 0)),
    ]
    return pl.pallas_call(
        _tc_layer_body, grid=grid, in_specs=specs,
        out_specs=pl.BlockSpec((_BN, dout), lambda i: (i, 0)),
        out_shape=jax.ShapeDtypeStruct((N, dout), jnp.float32),
    )(agg[:N], agg[NP:NP + N], deg[:N], deg[NP:NP + N], x, Wt, b2d)


def _tc_final_body(p0, p1, d0, d1, x, w3, b3, wp, bp, wc, bc,
                   emb, ne, ec):
    deg = jnp.maximum(d0[:, :1] + d1[:, :1], 1.0)
    t = (p0[...] + p1[...]) / deg + x[...]
    h = jnp.maximum(
        jnp.dot(t, w3[...], preferred_element_type=jnp.float32,
                precision=lax.Precision.HIGHEST) + b3[...], 0.0)
    emb[...] = h
    ne[...] = jnp.dot(h, wp[...], preferred_element_type=jnp.float32,
                      precision=lax.Precision.HIGHEST) + bp[...]
    ec[...] = jnp.dot(h, wc[...], preferred_element_type=jnp.float32,
                      precision=lax.Precision.HIGHEST) + bc[...]


def _tc_final(agg, deg, x, W3t, b3_2d, Wpt, bp2d, Wct, bc2d):
    grid = (N // _BN,)
    dh = W3t.shape[1]
    dp = Wpt.shape[1]
    dc = Wct.shape[1]
    specs = [
        pl.BlockSpec((_BN, D), lambda i: (i, 0)),
        pl.BlockSpec((_BN, D), lambda i: (i, 0)),
        pl.BlockSpec((_BN, D), lambda i: (i, 0)),
        pl.BlockSpec((_BN, D), lambda i: (i, 0)),
        pl.BlockSpec((_BN, D), lambda i: (i, 0)),
        pl.BlockSpec((D, dh), lambda i: (0, 0)),
        pl.BlockSpec((1, dh), lambda i: (0, 0)),
        pl.BlockSpec((dh, dp), lambda i: (0, 0)),
        pl.BlockSpec((1, dp), lambda i: (0, 0)),
        pl.BlockSpec((dh, dc), lambda i: (0, 0)),
        pl.BlockSpec((1, dc), lambda i: (0, 0)),
    ]
    return pl.pallas_call(
        _tc_final_body, grid=grid, in_specs=specs,
        out_specs=[
            pl.BlockSpec((_BN, dh), lambda i: (i, 0)),
            pl.BlockSpec((_BN, dp), lambda i: (i, 0)),
            pl.BlockSpec((_BN, dc), lambda i: (i, 0)),
        ],
        out_shape=[
            jax.ShapeDtypeStruct((N, dh), jnp.float32),
            jax.ShapeDtypeStruct((N, dp), jnp.float32),
            jax.ShapeDtypeStruct((N, dc), jnp.float32),
        ],
    )(agg[:N], agg[NP:NP + N], deg[:N], deg[NP:NP + N], x, W3t, b3_2d, Wpt,
      bp2d, Wct, bc2d)


def kernel(node_features, edge_index, W1, b1, W2, b2, W3, b3, Wp, bp, Wc, bc):
    npad = E2 - E
    pad_src = jnp.arange(npad, dtype=edge_index.dtype) % N
    row_p = jnp.concatenate([edge_index[0], pad_src])
    pad_dst = (N + jnp.arange(npad, dtype=edge_index.dtype) % (NP - N))
    col_p = jnp.concatenate([edge_index[1], pad_dst])
    row = row_p.reshape(NW * SBN, SB, CH)
    col = col_p.reshape(NW * SBN, SB, CH)
    z = jnp.zeros((CH, D), jnp.float32)
    ones = jnp.ones((CH, D), jnp.float32)

    (deg,) = _sc_deg(col, z, ones)
    (agg,) = _sc_agg(node_features, row, col, z)
    h1 = _tc_layer(agg, deg, node_features, W1.T, b1.reshape(1, -1), D)
    (agg,) = _sc_agg(h1, row, col, z)
    h2 = _tc_layer(agg, deg, h1, W2.T, b2.reshape(1, -1), D)
    (agg,) = _sc_agg(h2, row, col, z)

    n_cls = Wc.shape[0]
    Wct = jnp.zeros((W3.shape[0], 128), jnp.float32).at[:, :n_cls].set(Wc.T)
    bc_pad = jnp.zeros((1, 128), jnp.float32).at[:, :n_cls].set(bc)
    emb, ne, ec_pad = _tc_final(agg, deg, h2, W3.T, b3.reshape(1, -1),
                                Wp.T, bp.reshape(1, -1), Wct, bc_pad)
    return emb, ne, ec_pad[:, :n_cls]


# speedup vs baseline: 1.3675x; 1.0229x over previous
import jax
import jax.numpy as jnp
from jax import lax
from jax.experimental import pallas as pl
from jax.experimental.pallas import tpu as pltpu
from jax.experimental.pallas import tpu_sc as plsc

N = 10000
E = 320000
D = 128
NC = 2
NS = 16
NW = NC * NS
CH = 128
SB = 16
SBN = 5
E2 = NW * SBN * SB * CH
NP = 10240
RPT = NP // NS

_MESH = plsc.VectorSubcoreMesh(core_axis_name="c", subcore_axis_name="s",
                               num_cores=NC, num_subcores=NS)


def _sc_agg_body(x_hbm, row_hbm, col_hbm, z_hbm,
                 agg_out, row_v, col_v, buf0, buf1, acc_sh, sem0, sem1):
    c = lax.axis_index("c")
    s = lax.axis_index("s")
    wid = s * NC + c

    pltpu.sync_copy(z_hbm, buf0)

    @pl.loop(0, RPT // CH)
    def _(k):
        pltpu.sync_copy(buf0, acc_sh.at[pl.ds(s * RPT + k * CH, CH)])

    plsc.subcore_barrier()

    @pl.loop(0, SBN)
    def _(bk):
        pltpu.sync_copy(row_hbm.at[wid * SBN + bk], row_v)
        pltpu.sync_copy(col_hbm.at[wid * SBN + bk], col_v)

        pltpu.async_copy(x_hbm.at[row_v.at[0]], buf0, sem0)

        @pl.loop(0, SB // 2 - 1)
        def _(j):
            i = 2 * j
            pltpu.async_copy(x_hbm.at[row_v.at[i + 1]], buf1, sem1)
            pltpu.make_async_copy(x_hbm.at[row_v.at[i]], buf0, sem0).wait()
            pltpu.sync_copy(buf0, acc_sh.at[col_v.at[i]], add=True)
            pltpu.async_copy(x_hbm.at[row_v.at[i + 2]], buf0, sem0)
            pltpu.make_async_copy(x_hbm.at[row_v.at[i + 1]], buf1, sem1).wait()
            pltpu.sync_copy(buf1, acc_sh.at[col_v.at[i + 1]], add=True)

        pltpu.async_copy(x_hbm.at[row_v.at[SB - 1]], buf1, sem1)
        pltpu.make_async_copy(x_hbm.at[row_v.at[SB - 2]], buf0, sem0).wait()
        pltpu.sync_copy(buf0, acc_sh.at[col_v.at[SB - 2]], add=True)
        pltpu.make_async_copy(x_hbm.at[row_v.at[SB - 1]], buf1, sem1).wait()
        pltpu.sync_copy(buf1, acc_sh.at[col_v.at[SB - 1]], add=True)

    plsc.subcore_barrier()
    off = c * NP + s * RPT
    pltpu.sync_copy(acc_sh.at[pl.ds(s * RPT, RPT)], agg_out.at[pl.ds(off, RPT)])


_sc_agg = pl.kernel(
    _sc_agg_body,
    out_type=[jax.ShapeDtypeStruct((NC * NP, D), jnp.float32)],
    mesh=_MESH,
    scratch_types=[
        pltpu.VMEM((SB, CH), jnp.int32),
        pltpu.VMEM((SB, CH), jnp.int32),
        pltpu.VMEM((CH, D), jnp.float32),
        pltpu.VMEM((CH, D), jnp.float32),
        pltpu.VMEM_SHARED((NP, D), jnp.float32),
        pltpu.SemaphoreType.DMA,
        pltpu.SemaphoreType.DMA,
    ],
)


def _sc_deg_body(col_hbm, z_hbm, ones_hbm, deg_out, col_v, ones_v, rows_a,
                 acc_sh):
    c = lax.axis_index("c")
    s = lax.axis_index("s")
    wid = s * NC + c

    pltpu.sync_copy(z_hbm, rows_a)
    pltpu.sync_copy(ones_hbm, ones_v)

    @pl.loop(0, RPT // CH)
    def _(k):
        pltpu.sync_copy(rows_a, acc_sh.at[pl.ds(s * RPT + k * CH, CH)])

    plsc.subcore_barrier()

    @pl.loop(0, SBN)
    def _(bk):
        pltpu.sync_copy(col_hbm.at[wid * SBN + bk], col_v)

        @pl.loop(0, SB)
        def _(i):
            pltpu.sync_copy(ones_v, acc_sh.at[col_v.at[i]], add=True)

    plsc.subcore_barrier()
    off = c * NP + s * RPT
    pltpu.sync_copy(acc_sh.at[pl.ds(s * RPT, RPT)], deg_out.at[pl.ds(off, RPT)])


_sc_deg = pl.kernel(
    _sc_deg_body,
    out_type=[jax.ShapeDtypeStruct((NC * NP, D), jnp.float32)],
    mesh=_MESH,
    scratch_types=[
        pltpu.VMEM((SB, CH), jnp.int32),
        pltpu.VMEM((CH, D), jnp.float32),
        pltpu.VMEM((CH, D), jnp.float32),
        pltpu.VMEM_SHARED((NP, D), jnp.float32),
    ],
)

_BN = 2000


def _tc_layer_body(p0, p1, d0, d1, x, w, b, o):
    deg = jnp.maximum(d0[:, :1] + d1[:, :1], 1.0)
    t = (p0[...] + p1[...]) / deg + x[...]
    acc = jnp.dot(t, w[...], preferred_element_type=jnp.float32,
                  precision=lax.Precision.DEFAULT)
    o[...] = jnp.maximum(acc + b[...], 0.0)


def _tc_layer(agg, deg, x, Wt, b2d, dout):
    grid = (N // _BN,)
    specs = [
        pl.BlockSpec((_BN, D), lambda i: (i, 0)),
        pl.BlockSpec((_BN, D), lambda i: (i, 0)),
        pl.BlockSpec((_BN, D), lambda i: (i, 0)),
        pl.BlockSpec((_BN, D), lambda i: (i, 0)),
        pl.BlockSpec((_BN, D), lambda i: (i, 0)),
        pl.BlockSpec((D, dout), lambda i: (0, 0)),
        pl.BlockSpec((1, dout), lambda i: (0, 0)),
    ]
    return pl.pallas_call(
        _tc_layer_body, grid=grid, in_specs=specs,
        out_specs=pl.BlockSpec((_BN, dout), lambda i: (i, 0)),
        out_shape=jax.ShapeDtypeStruct((N, dout), jnp.float32),
    )(agg[:N], agg[NP:NP + N], deg[:N], deg[NP:NP + N], x, Wt, b2d)


def _tc_final_body(p0, p1, d0, d1, x, w3, b3, wp, bp, wc, bc,
                   emb, ne, ec):
    deg = jnp.maximum(d0[:, :1] + d1[:, :1], 1.0)
    t = (p0[...] + p1[...]) / deg + x[...]
    h = jnp.maximum(
        jnp.dot(t, w3[...], preferred_element_type=jnp.float32,
                precision=lax.Precision.DEFAULT) + b3[...], 0.0)
    emb[...] = h
    ne[...] = jnp.dot(h, wp[...], preferred_element_type=jnp.float32,
                      precision=lax.Precision.DEFAULT) + bp[...]
    ec[...] = jnp.dot(h, wc[...], preferred_element_type=jnp.float32,
                      precision=lax.Precision.DEFAULT) + bc[...]


def _tc_final(agg, deg, x, W3t, b3_2d, Wpt, bp2d, Wct, bc2d):
    grid = (N // _BN,)
    dh = W3t.shape[1]
    dp = Wpt.shape[1]
    dc = Wct.shape[1]
    specs = [
        pl.BlockSpec((_BN, D), lambda i: (i, 0)),
        pl.BlockSpec((_BN, D), lambda i: (i, 0)),
        pl.BlockSpec((_BN, D), lambda i: (i, 0)),
        pl.BlockSpec((_BN, D), lambda i: (i, 0)),
        pl.BlockSpec((_BN, D), lambda i: (i, 0)),
        pl.BlockSpec((D, dh), lambda i: (0, 0)),
        pl.BlockSpec((1, dh), lambda i: (0, 0)),
        pl.BlockSpec((dh, dp), lambda i: (0, 0)),
        pl.BlockSpec((1, dp), lambda i: (0, 0)),
        pl.BlockSpec((dh, dc), lambda i: (0, 0)),
        pl.BlockSpec((1, dc), lambda i: (0, 0)),
    ]
    return pl.pallas_call(
        _tc_final_body, grid=grid, in_specs=specs,
        out_specs=[
            pl.BlockSpec((_BN, dh), lambda i: (i, 0)),
            pl.BlockSpec((_BN, dp), lambda i: (i, 0)),
            pl.BlockSpec((_BN, dc), lambda i: (i, 0)),
        ],
        out_shape=[
            jax.ShapeDtypeStruct((N, dh), jnp.float32),
            jax.ShapeDtypeStruct((N, dp), jnp.float32),
            jax.ShapeDtypeStruct((N, dc), jnp.float32),
        ],
    )(agg[:N], agg[NP:NP + N], deg[:N], deg[NP:NP + N], x, W3t, b3_2d, Wpt,
      bp2d, Wct, bc2d)


def kernel(node_features, edge_index, W1, b1, W2, b2, W3, b3, Wp, bp, Wc, bc):
    npad = E2 - E
    pad_src = jnp.arange(npad, dtype=edge_index.dtype) % N
    row_p = jnp.concatenate([edge_index[0], pad_src])
    pad_dst = (N + jnp.arange(npad, dtype=edge_index.dtype) % (NP - N))
    col_p = jnp.concatenate([edge_index[1], pad_dst])
    row = row_p.reshape(NW * SBN, SB, CH)
    col = col_p.reshape(NW * SBN, SB, CH)
    z = jnp.zeros((CH, D), jnp.float32)
    ones = jnp.ones((CH, D), jnp.float32)

    (deg,) = _sc_deg(col, z, ones)
    (agg,) = _sc_agg(node_features, row, col, z)
    h1 = _tc_layer(agg, deg, node_features, W1.T, b1.reshape(1, -1), D)
    (agg,) = _sc_agg(h1, row, col, z)
    h2 = _tc_layer(agg, deg, h1, W2.T, b2.reshape(1, -1), D)
    (agg,) = _sc_agg(h2, row, col, z)

    n_cls = Wc.shape[0]
    Wct = jnp.zeros((W3.shape[0], 128), jnp.float32).at[:, :n_cls].set(Wc.T)
    bc_pad = jnp.zeros((1, 128), jnp.float32).at[:, :n_cls].set(bc)
    emb, ne, ec_pad = _tc_final(agg, deg, h2, W3.T, b3.reshape(1, -1),
                                Wp.T, bp.reshape(1, -1), Wct, bc_pad)
    return emb, ne, ec_pad[:, :n_cls]


# speedup vs baseline: 1.4171x; 1.0363x over previous
import jax
import jax.numpy as jnp
from jax import lax
from jax.experimental import pallas as pl
from jax.experimental.pallas import tpu as pltpu
from jax.experimental.pallas import tpu_sc as plsc

N = 10000
E = 320000
D = 128
NC = 2
NS = 16
NW = NC * NS
CH = 128
SB = 16
SBN = 5
E2 = NW * SBN * SB * CH
NP = 10240
RPT = NP // NS

_MESH = plsc.VectorSubcoreMesh(core_axis_name="c", subcore_axis_name="s",
                               num_cores=NC, num_subcores=NS)


def _sc_agg_body(x_hbm, row_hbm, col_hbm, z_hbm, agg_out,
                 row_v0, col_v0, row_v1, col_v1, buf0, buf1, acc_sh,
                 sem0, sem1):
    c = lax.axis_index("c")
    s = lax.axis_index("s")
    wid = s * NC + c

    pltpu.sync_copy(z_hbm, buf0)

    @pl.loop(0, RPT // CH)
    def _(k):
        pltpu.sync_copy(buf0, acc_sh.at[pl.ds(s * RPT + k * CH, CH)])

    plsc.subcore_barrier()

    idx = [(row_v0, col_v0), (row_v1, col_v1)]
    pltpu.sync_copy(row_hbm.at[wid * SBN], row_v0)
    pltpu.sync_copy(col_hbm.at[wid * SBN], col_v0)
    pltpu.async_copy(x_hbm.at[row_v0.at[0]], buf0, sem0)

    for bk in range(SBN):
        rv, cv = idx[bk % 2]
        rvn, cvn = idx[(bk + 1) % 2]
        if bk + 1 < SBN:
            pltpu.sync_copy(row_hbm.at[wid * SBN + bk + 1], rvn)
            pltpu.sync_copy(col_hbm.at[wid * SBN + bk + 1], cvn)

        @pl.loop(0, SB // 2 - 1)
        def _(j, rv=rv, cv=cv):
            i = 2 * j
            pltpu.async_copy(x_hbm.at[rv.at[i + 1]], buf1, sem1)
            pltpu.make_async_copy(x_hbm.at[rv.at[i]], buf0, sem0).wait()
            pltpu.sync_copy(buf0, acc_sh.at[cv.at[i]], add=True)
            pltpu.async_copy(x_hbm.at[rv.at[i + 2]], buf0, sem0)
            pltpu.make_async_copy(x_hbm.at[rv.at[i + 1]], buf1, sem1).wait()
            pltpu.sync_copy(buf1, acc_sh.at[cv.at[i + 1]], add=True)

        pltpu.async_copy(x_hbm.at[rv.at[SB - 1]], buf1, sem1)
        pltpu.make_async_copy(x_hbm.at[rv.at[SB - 2]], buf0, sem0).wait()
        pltpu.sync_copy(buf0, acc_sh.at[cv.at[SB - 2]], add=True)
        if bk + 1 < SBN:
            pltpu.async_copy(x_hbm.at[rvn.at[0]], buf0, sem0)
        pltpu.make_async_copy(x_hbm.at[rv.at[SB - 1]], buf1, sem1).wait()
        pltpu.sync_copy(buf1, acc_sh.at[cv.at[SB - 1]], add=True)

    plsc.subcore_barrier()
    off = c * NP + s * RPT
    pltpu.sync_copy(acc_sh.at[pl.ds(s * RPT, RPT)], agg_out.at[pl.ds(off, RPT)])


_sc_agg = pl.kernel(
    _sc_agg_body,
    out_type=[jax.ShapeDtypeStruct((NC * NP, D), jnp.float32)],
    mesh=_MESH,
    scratch_types=[
        pltpu.VMEM((SB, CH), jnp.int32),
        pltpu.VMEM((SB, CH), jnp.int32),
        pltpu.VMEM((SB, CH), jnp.int32),
        pltpu.VMEM((SB, CH), jnp.int32),
        pltpu.VMEM((CH, D), jnp.float32),
        pltpu.VMEM((CH, D), jnp.float32),
        pltpu.VMEM_SHARED((NP, D), jnp.float32),
        pltpu.SemaphoreType.DMA,
        pltpu.SemaphoreType.DMA,
    ],
)


def _sc_deg_body(col_hbm, z_hbm, ones_hbm, deg_out, col_v, ones_v, rows_a,
                 acc_sh):
    c = lax.axis_index("c")
    s = lax.axis_index("s")
    wid = s * NC + c

    pltpu.sync_copy(z_hbm, rows_a)
    pltpu.sync_copy(ones_hbm, ones_v)

    @pl.loop(0, RPT // CH)
    def _(k):
        pltpu.sync_copy(rows_a, acc_sh.at[pl.ds(s * RPT + k * CH, CH)])

    plsc.subcore_barrier()

    @pl.loop(0, SBN)
    def _(bk):
        pltpu.sync_copy(col_hbm.at[wid * SBN + bk], col_v)

        @pl.loop(0, SB)
        def _(i):
            pltpu.sync_copy(ones_v, acc_sh.at[col_v.at[i]], add=True)

    plsc.subcore_barrier()
    off = c * NP + s * RPT
    pltpu.sync_copy(acc_sh.at[pl.ds(s * RPT, RPT)], deg_out.at[pl.ds(off, RPT)])


_sc_deg = pl.kernel(
    _sc_deg_body,
    out_type=[jax.ShapeDtypeStruct((NC * NP, D), jnp.float32)],
    mesh=_MESH,
    scratch_types=[
        pltpu.VMEM((SB, CH), jnp.int32),
        pltpu.VMEM((CH, D), jnp.float32),
        pltpu.VMEM((CH, D), jnp.float32),
        pltpu.VMEM_SHARED((NP, D), jnp.float32),
    ],
)

_BN = 2000


def _tc_layer_body(p0, p1, d0, d1, x, w, b, o):
    deg = jnp.maximum(d0[:, :1] + d1[:, :1], 1.0)
    t = (p0[...] + p1[...]) / deg + x[...]
    acc = jnp.dot(t, w[...], preferred_element_type=jnp.float32,
                  precision=lax.Precision.DEFAULT)
    o[...] = jnp.maximum(acc + b[...], 0.0)


def _tc_layer(agg, deg, x, Wt, b2d, dout):
    grid = (N // _BN,)
    specs = [
        pl.BlockSpec((_BN, D), lambda i: (i, 0)),
        pl.BlockSpec((_BN, D), lambda i: (i, 0)),
        pl.BlockSpec((_BN, D), lambda i: (i, 0)),
        pl.BlockSpec((_BN, D), lambda i: (i, 0)),
        pl.BlockSpec((_BN, D), lambda i: (i, 0)),
        pl.BlockSpec((D, dout), lambda i: (0, 0)),
        pl.BlockSpec((1, dout), lambda i: (0, 0)),
    ]
    return pl.pallas_call(
        _tc_layer_body, grid=grid, in_specs=specs,
        out_specs=pl.BlockSpec((_BN, dout), lambda i: (i, 0)),
        out_shape=jax.ShapeDtypeStruct((N, dout), jnp.float32),
    )(agg[:N], agg[NP:NP + N], deg[:N], deg[NP:NP + N], x, Wt, b2d)


def _tc_final_body(p0, p1, d0, d1, x, w3, b3, wp, bp, wc, bc,
                   emb, ne, ec):
    deg = jnp.maximum(d0[:, :1] + d1[:, :1], 1.0)
    t = (p0[...] + p1[...]) / deg + x[...]
    h = jnp.maximum(
        jnp.dot(t, w3[...], preferred_element_type=jnp.float32,
                precision=lax.Precision.DEFAULT) + b3[...], 0.0)
    emb[...] = h
    ne[...] = jnp.dot(h, wp[...], preferred_element_type=jnp.float32,
                      precision=lax.Precision.DEFAULT) + bp[...]
    ec[...] = jnp.dot(h, wc[...], preferred_element_type=jnp.float32,
                      precision=lax.Precision.DEFAULT) + bc[...]


def _tc_final(agg, deg, x, W3t, b3_2d, Wpt, bp2d, Wct, bc2d):
    grid = (N // _BN,)
    dh = W3t.shape[1]
    dp = Wpt.shape[1]
    dc = Wct.shape[1]
    specs = [
        pl.BlockSpec((_BN, D), lambda i: (i, 0)),
        pl.BlockSpec((_BN, D), lambda i: (i, 0)),
        pl.BlockSpec((_BN, D), lambda i: (i, 0)),
        pl.BlockSpec((_BN, D), lambda i: (i, 0)),
        pl.BlockSpec((_BN, D), lambda i: (i, 0)),
        pl.BlockSpec((D, dh), lambda i: (0, 0)),
        pl.BlockSpec((1, dh), lambda i: (0, 0)),
        pl.BlockSpec((dh, dp), lambda i: (0, 0)),
        pl.BlockSpec((1, dp), lambda i: (0, 0)),
        pl.BlockSpec((dh, dc), lambda i: (0, 0)),
        pl.BlockSpec((1, dc), lambda i: (0, 0)),
    ]
    return pl.pallas_call(
        _tc_final_body, grid=grid, in_specs=specs,
        out_specs=[
            pl.BlockSpec((_BN, dh), lambda i: (i, 0)),
            pl.BlockSpec((_BN, dp), lambda i: (i, 0)),
            pl.BlockSpec((_BN, dc), lambda i: (i, 0)),
        ],
        out_shape=[
            jax.ShapeDtypeStruct((N, dh), jnp.float32),
            jax.ShapeDtypeStruct((N, dp), jnp.float32),
            jax.ShapeDtypeStruct((N, dc), jnp.float32),
        ],
    )(agg[:N], agg[NP:NP + N], deg[:N], deg[NP:NP + N], x, W3t, b3_2d, Wpt,
      bp2d, Wct, bc2d)


def kernel(node_features, edge_index, W1, b1, W2, b2, W3, b3, Wp, bp, Wc, bc):
    npad = E2 - E
    pad_src = jnp.arange(npad, dtype=edge_index.dtype) % N
    row_p = jnp.concatenate([edge_index[0], pad_src])
    pad_dst = (N + jnp.arange(npad, dtype=edge_index.dtype) % (NP - N))
    col_p = jnp.concatenate([edge_index[1], pad_dst])
    row = row_p.reshape(NW * SBN, SB, CH)
    col = col_p.reshape(NW * SBN, SB, CH)
    z = jnp.zeros((CH, D), jnp.float32)
    ones = jnp.ones((CH, D), jnp.float32)

    (deg,) = _sc_deg(col, z, ones)
    (agg,) = _sc_agg(node_features, row, col, z)
    h1 = _tc_layer(agg, deg, node_features, W1.T, b1.reshape(1, -1), D)
    (agg,) = _sc_agg(h1, row, col, z)
    h2 = _tc_layer(agg, deg, h1, W2.T, b2.reshape(1, -1), D)
    (agg,) = _sc_agg(h2, row, col, z)

    n_cls = Wc.shape[0]
    Wct = jnp.zeros((W3.shape[0], 128), jnp.float32).at[:, :n_cls].set(Wc.T)
    bc_pad = jnp.zeros((1, 128), jnp.float32).at[:, :n_cls].set(bc)
    emb, ne, ec_pad = _tc_final(agg, deg, h2, W3.T, b3.reshape(1, -1),
                                Wp.T, bp.reshape(1, -1), Wct, bc_pad)
    return emb, ne, ec_pad[:, :n_cls]
